# baseline (speedup 1.0000x reference)
import functools

import jax
import jax.numpy as jnp
from jax import lax
from jax.experimental import pallas as pl
from jax.experimental.pallas import tpu as pltpu
from jax.experimental.pallas import tpu_sc as plsc

N = 10000
E = 320000
D = 128

NC = 2
NS = 16
NW = NC * NS

EPW = E // NW
CHUNK = 80
SUPER = 8
NCHUNK_P = 128
NSUP = NCHUNK_P // SUPER
EPW_P = NCHUNK_P * CHUNK


def _spmm_sc():
    mesh = plsc.VectorSubcoreMesh(core_axis_name="c", subcore_axis_name="s")

    @functools.partial(
        pl.kernel,
        mesh=mesh,
        out_type=jax.ShapeDtypeStruct((NC, N, D), jnp.float32),
        scratch_types=[
            pltpu.VMEM((2, SUPER, CHUNK), jnp.int32),
            pltpu.VMEM((2, SUPER, CHUNK), jnp.int32),
            pltpu.VMEM((2, SUPER, CHUNK), jnp.float32),
            pltpu.VMEM((2, CHUNK, D), jnp.float32),
            pltpu.VMEM_SHARED((N, D), jnp.float32),
            pltpu.SemaphoreType.DMA,
            pltpu.SemaphoreType.DMA,
            pltpu.SemaphoreType.DMA,
        ],
    )
    def k(x_hbm, col_hbm, row_hbm, w_hbm, zero_hbm, out_hbm,
          col_v, row_v, w_v, rows_v, acc_sh, isem, g0, g1):
        cid = lax.axis_index("c")
        sid = lax.axis_index("s")
        wid = sid * NC + cid
        gsems = (g0, g1)

        z0 = sid * 624
        pltpu.sync_copy(zero_hbm.at[pl.ds(z0, 624)], acc_sh.at[pl.ds(z0, 624)])

        @pl.when(sid == NS - 1)
        def _():
            pltpu.sync_copy(zero_hbm.at[pl.ds(16 * 624, N - 16 * 624)],
                            acc_sh.at[pl.ds(16 * 624, N - 16 * 624)])

        def idx_copies(s_next, q):
            sl = pl.ds(s_next * SUPER, SUPER)
            return (
                pltpu.make_async_copy(col_hbm.at[wid, sl], col_v.at[q], isem),
                pltpu.make_async_copy(row_hbm.at[wid, sl], row_v.at[q], isem),
                pltpu.make_async_copy(w_hbm.at[wid, sl], w_v.at[q], isem),
            )

        def gather(p, j, b):
            return pltpu.make_async_copy(
                x_hbm.at[col_v.at[p, j]], rows_v.at[b], gsems[b])

        for c in idx_copies(0, 0):
            c.start()
        for c in idx_copies(0, 0):
            c.wait()
        gather(0, 0, 0).start()

        plsc.subcore_barrier()

        def scale(p, j, b):
            def grp(g, c2):
                wvec = w_v[p, j, pl.ds(g * 16, 16)]
                for l in range(16):
                    wl = wvec[l]
                    e = g * 16 + l
                    for jj in range(D // 16):
                        sl = pl.ds(jj * 16, 16)
                        rows_v[b, e, sl] = rows_v[b, e, sl] * wl
                return c2

            lax.fori_loop(0, CHUNK // 16, grp, 0)

        def super_body(s, carry):
            p = lax.rem(s, 2)
            q = 1 - p
            for j in range(SUPER):
                b = j % 2
                if j == 1:
                    @pl.when(s < NSUP - 1)
                    def _():
                        for c in idx_copies(s + 1, q):
                            c.start()
                if j < SUPER - 1:
                    gather(p, j + 1, 1 - b).start()
                else:
                    @pl.when(s < NSUP - 1)
                    def _():
                        for c in idx_copies(s + 1, q):
                            c.wait()
                        gather(q, 0, 1 - b).start()
                gather(p, j, b).wait()
                scale(p, j, b)
                pltpu.sync_copy(rows_v.at[b], acc_sh.at[row_v.at[p, j]],
                                add=True)
            return carry

        lax.fori_loop(0, NSUP, super_body, 0)

        plsc.subcore_barrier()

        r0 = sid * 624
        pltpu.sync_copy(acc_sh.at[pl.ds(r0, 624)],
                        out_hbm.at[cid, pl.ds(r0, 624)])

        @pl.when(sid == NS - 1)
        def _():
            pltpu.sync_copy(acc_sh.at[pl.ds(16 * 624, N - 16 * 624)],
                            out_hbm.at[cid, pl.ds(16 * 624, N - 16 * 624)])

    return k


def _combine_kernel(a_ref, b_ref, o_ref):
    o_ref[...] = a_ref[...] + b_ref[...]


_BLK = 1000


def _combine(partials):
    grid = (N // _BLK,)
    return pl.pallas_call(
        _combine_kernel,
        grid=grid,
        in_specs=[pl.BlockSpec((_BLK, D), lambda i: (i, 0)),
                  pl.BlockSpec((_BLK, D), lambda i: (i, 0))],
        out_specs=pl.BlockSpec((_BLK, D), lambda i: (i, 0)),
        out_shape=jax.ShapeDtypeStruct((N, D), jnp.float32),
    )(partials[0], partials[1])


def _pad_edges(a):
    a = a.reshape(NW, EPW)
    return jnp.pad(a, ((0, 0), (0, EPW_P - EPW))).reshape(
        NW, NCHUNK_P, CHUNK)


@jax.jit
def kernel(x, edge_index, edge_weight):
    row = _pad_edges(edge_index[0])
    col = _pad_edges(edge_index[1])
    w = _pad_edges(edge_weight)
    zeros = jnp.zeros((N, D), jnp.float32)
    partials = _spmm_sc()(x, col, row, w, zeros)
    return _combine(partials)


# speedup vs baseline: 1.0690x; 1.0690x over previous
import functools

import jax
import jax.numpy as jnp
from jax import lax
from jax.experimental import pallas as pl
from jax.experimental.pallas import tpu as pltpu
from jax.experimental.pallas import tpu_sc as plsc

N = 10000
E = 320000
D = 128

NC = 2
NS = 16
NW = NC * NS

EPW = E // NW
CHUNK = 80
NCHUNK = EPW // CHUNK


def _spmm_sc():
    mesh = plsc.VectorSubcoreMesh(core_axis_name="c", subcore_axis_name="s")

    @functools.partial(
        pl.kernel,
        mesh=mesh,
        out_type=jax.ShapeDtypeStruct((NC, N, D), jnp.float32),
        scratch_types=[
            pltpu.VMEM((CHUNK,), jnp.int32),
            pltpu.VMEM((CHUNK,), jnp.int32),
            pltpu.VMEM((CHUNK,), jnp.float32),
            pltpu.VMEM((CHUNK, D), jnp.float32),
            pltpu.VMEM_SHARED((N, D), jnp.float32),
            pltpu.SemaphoreType.DMA,
        ],
    )
    def k(x_hbm, col_hbm, row_hbm, w_hbm, zero_hbm, out_hbm,
          col_v, row_v, w_v, rows_v, acc_sh, sem):
        cid = lax.axis_index("c")
        sid = lax.axis_index("s")
        wid = sid * NC + cid

        z0 = sid * 624
        pltpu.sync_copy(zero_hbm.at[pl.ds(z0, 624)], acc_sh.at[pl.ds(z0, 624)])

        @pl.when(sid == NS - 1)
        def _():
            pltpu.sync_copy(zero_hbm.at[pl.ds(16 * 624, N - 16 * 624)],
                            acc_sh.at[pl.ds(16 * 624, N - 16 * 624)])

        plsc.subcore_barrier()

        base = wid * EPW

        def chunk_body(i, carry):
            off = base + i * CHUNK
            pltpu.sync_copy(col_hbm.at[pl.ds(off, CHUNK)], col_v)
            pltpu.sync_copy(row_hbm.at[pl.ds(off, CHUNK)], row_v)
            pltpu.sync_copy(w_hbm.at[pl.ds(off, CHUNK)], w_v)
            pltpu.async_copy(x_hbm.at[col_v], rows_v, sem).wait()

            def scale_body(g, carry2):
                wvec = w_v[pl.ds(g * 16, 16)]
                for l in range(16):
                    w = wvec[l]
                    e = g * 16 + l
                    for j in range(D // 16):
                        sl = pl.ds(j * 16, 16)
                        rows_v[e, sl] = rows_v[e, sl] * w
                return carry2

            lax.fori_loop(0, CHUNK // 16, scale_body, 0)

            pltpu.sync_copy(rows_v, acc_sh.at[row_v], add=True)
            return carry

        lax.fori_loop(0, NCHUNK, chunk_body, 0)

        plsc.subcore_barrier()

        r0 = sid * 624
        pltpu.sync_copy(acc_sh.at[pl.ds(r0, 624)],
                        out_hbm.at[cid, pl.ds(r0, 624)])

        @pl.when(sid == NS - 1)
        def _():
            pltpu.sync_copy(acc_sh.at[pl.ds(16 * 624, N - 16 * 624)],
                            out_hbm.at[cid, pl.ds(16 * 624, N - 16 * 624)])

    return k


def _combine_kernel(a_ref, b_ref, o_ref):
    o_ref[...] = a_ref[...] + b_ref[...]


_BLK = 1000


def _combine(partials):
    grid = (N // _BLK,)
    return pl.pallas_call(
        _combine_kernel,
        grid=grid,
        in_specs=[pl.BlockSpec((_BLK, D), lambda i: (i, 0)),
                  pl.BlockSpec((_BLK, D), lambda i: (i, 0))],
        out_specs=pl.BlockSpec((_BLK, D), lambda i: (i, 0)),
        out_shape=jax.ShapeDtypeStruct((N, D), jnp.float32),
    )(partials[0], partials[1])


@jax.jit
def kernel(x, edge_index, edge_weight):
    row = edge_index[0]
    col = edge_index[1]
    zeros = jnp.zeros((N, D), jnp.float32)
    partials = _spmm_sc()(x, col, row, edge_weight, zeros)
    return _combine(partials)
